# baseline (speedup 1.0000x reference)
import jax
import jax.numpy as jnp
from jax.experimental import pallas as pl

NUM_EMBEDDINGS = 256
EMBEDDING_DIM = 128
BLOCK_M = 2048


def _split3(cb):
    hi = cb.astype(jnp.bfloat16)
    rem = cb - hi.astype(jnp.float32)
    mid = rem.astype(jnp.bfloat16)
    lo = (rem - mid.astype(jnp.float32)).astype(jnp.bfloat16)
    return hi, mid, lo


def _bf16_dot(a, b):
    return jax.lax.dot_general(
        a, b, (((1,), (0,)), ((), ())),
        preferred_element_type=jnp.float32)


def _rvq_body(x_ref, cb0_ref, cb1_ref, o_ref):
    xb = x_ref[...]
    cb0 = cb0_ref[...]
    cb1 = cb1_ref[...]

    lane = jax.lax.broadcasted_iota(jnp.int32, (xb.shape[0], NUM_EMBEDDINGS), 1)

    def pick(res, cb):
        r2 = jnp.sum(res * res, axis=1, keepdims=True)
        c2 = jnp.sum(cb * cb, axis=1)
        cross = jax.lax.dot_general(
            res, cb, (((1,), (1,)), ((), ())),
            preferred_element_type=jnp.float32)
        d = (r2 - 2.0 * cross) + c2[None, :]
        idx = jnp.argmin(d, axis=1).astype(jnp.int32)
        return (lane == idx[:, None]).astype(jnp.bfloat16)

    oh0 = pick(xb, cb0)
    hi0, mid0, lo0 = _split3(cb0)
    parts = _bf16_dot(oh0, jnp.concatenate([hi0, mid0, lo0], axis=1))
    q0 = ((parts[:, :EMBEDDING_DIM] + parts[:, EMBEDDING_DIM:2 * EMBEDDING_DIM])
          + parts[:, 2 * EMBEDDING_DIM:])

    res1 = xb - q0
    oh1 = pick(res1, cb1)
    q1 = _bf16_dot(oh1, cb1.astype(jnp.bfloat16))

    o_ref[...] = xb + ((q0 + q1) - xb)


def kernel(x, codebook0, codebook1):
    b, n, d = x.shape
    m_total = b * n
    x2 = x.reshape(m_total, d)
    grid = (m_total // BLOCK_M,)
    out = pl.pallas_call(
        _rvq_body,
        grid=grid,
        in_specs=[
            pl.BlockSpec((BLOCK_M, d), lambda i: (i, 0)),
            pl.BlockSpec((NUM_EMBEDDINGS, d), lambda i: (0, 0)),
            pl.BlockSpec((NUM_EMBEDDINGS, d), lambda i: (0, 0)),
        ],
        out_specs=pl.BlockSpec((BLOCK_M, d), lambda i: (i, 0)),
        out_shape=jax.ShapeDtypeStruct((m_total, d), jnp.float32),
    )(x2, codebook0, codebook1)
    return out.reshape(b, n, d)


# speedup vs baseline: 1.4919x; 1.4919x over previous
import jax
import jax.numpy as jnp
from jax.experimental import pallas as pl

NUM_EMBEDDINGS = 256
EMBEDDING_DIM = 128
BLOCK_M = 2048


def _split3(cb):
    hi = cb.astype(jnp.bfloat16)
    rem = cb - hi.astype(jnp.float32)
    mid = rem.astype(jnp.bfloat16)
    lo = (rem - mid.astype(jnp.float32)).astype(jnp.bfloat16)
    return hi, mid, lo


def _bf16_dot(a, b):
    return jax.lax.dot_general(
        a, b, (((1,), (0,)), ((), ())),
        preferred_element_type=jnp.float32)


def _rvq_body(x_ref, cb0_ref, cb1_ref, o_ref):
    xb = x_ref[...]
    cb0 = cb0_ref[...]
    cb1 = cb1_ref[...]

    lane = jax.lax.broadcasted_iota(jnp.int32, (xb.shape[0], NUM_EMBEDDINGS), 1)

    def pick(res, cb):
        r2 = jnp.sum(res * res, axis=1, keepdims=True)
        c2 = jnp.sum(cb * cb, axis=1)
        cross = jax.lax.dot_general(
            res, cb, (((1,), (1,)), ((), ())),
            preferred_element_type=jnp.float32)
        d = (r2 - 2.0 * cross) + c2[None, :]
        m = jnp.min(d, axis=1, keepdims=True)
        idx = jnp.min(jnp.where(d == m, lane, NUM_EMBEDDINGS), axis=1)
        return (lane == idx[:, None]).astype(jnp.bfloat16)

    oh0 = pick(xb, cb0)
    hi0, mid0, lo0 = _split3(cb0)
    parts = _bf16_dot(oh0, jnp.concatenate([hi0, mid0, lo0], axis=1))
    q0 = ((parts[:, :EMBEDDING_DIM] + parts[:, EMBEDDING_DIM:2 * EMBEDDING_DIM])
          + parts[:, 2 * EMBEDDING_DIM:])

    res1 = xb - q0
    oh1 = pick(res1, cb1)
    q1 = _bf16_dot(oh1, cb1.astype(jnp.bfloat16))

    o_ref[...] = xb + ((q0 + q1) - xb)


def kernel(x, codebook0, codebook1):
    b, n, d = x.shape
    m_total = b * n
    x2 = x.reshape(m_total, d)
    grid = (m_total // BLOCK_M,)
    out = pl.pallas_call(
        _rvq_body,
        grid=grid,
        in_specs=[
            pl.BlockSpec((BLOCK_M, d), lambda i: (i, 0)),
            pl.BlockSpec((NUM_EMBEDDINGS, d), lambda i: (0, 0)),
            pl.BlockSpec((NUM_EMBEDDINGS, d), lambda i: (0, 0)),
        ],
        out_specs=pl.BlockSpec((BLOCK_M, d), lambda i: (i, 0)),
        out_shape=jax.ShapeDtypeStruct((m_total, d), jnp.float32),
    )(x2, codebook0, codebook1)
    return out.reshape(b, n, d)


# speedup vs baseline: 1.5956x; 1.0695x over previous
import jax
import jax.numpy as jnp
from jax.experimental import pallas as pl

NUM_EMBEDDINGS = 256
EMBEDDING_DIM = 128
BLOCK_M = 2048


def _split3(cb):
    hi = cb.astype(jnp.bfloat16)
    rem = cb - hi.astype(jnp.float32)
    mid = rem.astype(jnp.bfloat16)
    lo = (rem - mid.astype(jnp.float32)).astype(jnp.bfloat16)
    return hi, mid, lo


def _bf16_dot(a, b):
    return jax.lax.dot_general(
        a, b, (((1,), (0,)), ((), ())),
        preferred_element_type=jnp.float32)


def _rvq_body(x_ref, cb0_ref, cb1_ref, o_ref):
    xb = x_ref[...]
    cb0 = cb0_ref[...]
    cb1 = cb1_ref[...]

    lane = jax.lax.broadcasted_iota(
        jnp.int32, (xb.shape[0], NUM_EMBEDDINGS), 1).astype(jnp.float32)

    def pick(res, cb):
        r2 = jnp.sum(res * res, axis=1, keepdims=True)
        c2 = jnp.sum(cb * cb, axis=1)
        cross = jax.lax.dot_general(
            res, cb, (((1,), (1,)), ((), ())),
            preferred_element_type=jnp.float32)
        d = (r2 - 2.0 * cross) + c2[None, :]
        m = jnp.min(d, axis=1, keepdims=True)
        idx = jnp.min(jnp.where(d == m, lane, float(NUM_EMBEDDINGS)),
                      axis=1, keepdims=True)
        return (lane == idx).astype(jnp.bfloat16)

    oh0 = pick(xb, cb0)
    hi0, mid0, lo0 = _split3(cb0)
    parts = _bf16_dot(oh0, jnp.concatenate([hi0, mid0, lo0], axis=1))
    q0 = ((parts[:, :EMBEDDING_DIM] + parts[:, EMBEDDING_DIM:2 * EMBEDDING_DIM])
          + parts[:, 2 * EMBEDDING_DIM:])

    res1 = xb - q0
    oh1 = pick(res1, cb1)
    q1 = _bf16_dot(oh1, cb1.astype(jnp.bfloat16))

    o_ref[...] = xb + ((q0 + q1) - xb)


def kernel(x, codebook0, codebook1):
    b, n, d = x.shape
    m_total = b * n
    x2 = x.reshape(m_total, d)
    grid = (m_total // BLOCK_M,)
    out = pl.pallas_call(
        _rvq_body,
        grid=grid,
        in_specs=[
            pl.BlockSpec((BLOCK_M, d), lambda i: (i, 0)),
            pl.BlockSpec((NUM_EMBEDDINGS, d), lambda i: (0, 0)),
            pl.BlockSpec((NUM_EMBEDDINGS, d), lambda i: (0, 0)),
        ],
        out_specs=pl.BlockSpec((BLOCK_M, d), lambda i: (i, 0)),
        out_shape=jax.ShapeDtypeStruct((m_total, d), jnp.float32),
    )(x2, codebook0, codebook1)
    return out.reshape(b, n, d)


# speedup vs baseline: 2.0098x; 1.2596x over previous
import jax
import jax.numpy as jnp
from jax.experimental import pallas as pl

NUM_EMBEDDINGS = 256
EMBEDDING_DIM = 128
BLOCK_M = 2048


def _split3(cb):
    hi = cb.astype(jnp.bfloat16)
    rem = cb - hi.astype(jnp.float32)
    mid = rem.astype(jnp.bfloat16)
    lo = (rem - mid.astype(jnp.float32)).astype(jnp.bfloat16)
    return hi, mid, lo


def _bf16_dot(a, b):
    return jax.lax.dot_general(
        a, b, (((1,), (0,)), ((), ())),
        preferred_element_type=jnp.float32)


def _rvq_body(x_ref, cb0_ref, cb1_ref, o_ref):
    cb0 = cb0_ref[...]
    cb1 = cb1_ref[...]

    half = BLOCK_M // 2
    lane = jax.lax.broadcasted_iota(
        jnp.int32, (half, NUM_EMBEDDINGS), 1).astype(jnp.float32)

    hi0, mid0, lo0 = _split3(cb0)
    cb0_parts = jnp.concatenate([hi0, mid0, lo0], axis=1)
    cb1_bf = cb1.astype(jnp.bfloat16)
    c2_0 = jnp.sum(cb0 * cb0, axis=1)
    c2_1 = jnp.sum(cb1 * cb1, axis=1)

    def pick(res, cb, c2):
        r2 = jnp.sum(res * res, axis=1, keepdims=True)
        cross = jax.lax.dot_general(
            res, cb, (((1,), (1,)), ((), ())),
            preferred_element_type=jnp.float32)
        d = (r2 - 2.0 * cross) + c2[None, :]
        m = jnp.min(d, axis=1, keepdims=True)
        idx = jnp.min(jnp.where(d == m, lane, float(NUM_EMBEDDINGS)),
                      axis=1, keepdims=True)
        return (lane == idx).astype(jnp.bfloat16)

    def rvq(xb):
        oh0 = pick(xb, cb0, c2_0)
        parts = _bf16_dot(oh0, cb0_parts)
        q0 = ((parts[:, :EMBEDDING_DIM]
               + parts[:, EMBEDDING_DIM:2 * EMBEDDING_DIM])
              + parts[:, 2 * EMBEDDING_DIM:])
        res1 = xb - q0
        oh1 = pick(res1, cb1, c2_1)
        q1 = _bf16_dot(oh1, cb1_bf)
        return xb + ((q0 + q1) - xb)

    xa = x_ref[:half, :]
    xc = x_ref[half:, :]
    o_ref[:half, :] = rvq(xa)
    o_ref[half:, :] = rvq(xc)


def kernel(x, codebook0, codebook1):
    b, n, d = x.shape
    m_total = b * n
    x2 = x.reshape(m_total, d)
    grid = (m_total // BLOCK_M,)
    out = pl.pallas_call(
        _rvq_body,
        grid=grid,
        in_specs=[
            pl.BlockSpec((BLOCK_M, d), lambda i: (i, 0)),
            pl.BlockSpec((NUM_EMBEDDINGS, d), lambda i: (0, 0)),
            pl.BlockSpec((NUM_EMBEDDINGS, d), lambda i: (0, 0)),
        ],
        out_specs=pl.BlockSpec((BLOCK_M, d), lambda i: (i, 0)),
        out_shape=jax.ShapeDtypeStruct((m_total, d), jnp.float32),
    )(x2, codebook0, codebook1)
    return out.reshape(b, n, d)
